# baseline (speedup 1.0000x reference)
import jax
import jax.numpy as jnp
from jax.experimental import pallas as pl

_B = 4
_L = 1024
_H = 4
_D = 32
_BS = 32
_S = 4
_T = _B * _L
_NB = _L // _BS
_SCALE = _D ** (-0.5)


def _silu(x):
    return x * jax.nn.sigmoid(x)


def _fwd(q_ref, k_ref, v_ref, gc_ref, gs_ref, tri_ref, o_ref):
    qt = q_ref[0, 0]
    kk = k_ref[0, 0]
    vv = v_ref[0, 0]
    gc = gc_ref[0, 0, 0]
    gs = gs_ref[0, 0, 0]
    tri = tri_ref[0, 0]

    blk_of_col = jax.lax.broadcasted_iota(jnp.int32, (_NB, _L), 1) // _BS
    blk_row = jax.lax.broadcasted_iota(jnp.int32, (_NB, _L), 0)
    expand = (blk_of_col == blk_row).astype(jnp.float32)

    k_cmp = kk.reshape(_NB, _BS, _D).sum(axis=1) * (1.0 / _BS)
    v_cmp = vv.reshape(_NB, _BS, _D).sum(axis=1) * (1.0 / _BS)

    s_cmp = jax.lax.dot_general(
        qt, k_cmp, (((1,), (1,)), ((), ())),
        preferred_element_type=jnp.float32) * _SCALE
    row = jax.lax.broadcasted_iota(jnp.int32, (_L, _NB), 0)
    col = jax.lax.broadcasted_iota(jnp.int32, (_L, _NB), 1)
    blk_causal = (row // _BS) >= col
    p_cmp = jnp.where(blk_causal, _silu(s_cmp), 0.0)
    o_cmp = jnp.dot(p_cmp, v_cmp, preferred_element_type=jnp.float32) * gc

    neginf = jnp.float32(-jnp.inf)
    work = jnp.where(blk_causal, s_cmp, neginf)
    sel = jnp.zeros((_L, _NB), dtype=jnp.bool_)
    for _ in range(_S):
        m = jnp.max(work, axis=1, keepdims=True)
        ismax = jnp.logical_and(work == m, jnp.logical_not(sel))
        cand = jnp.where(ismax, col, _NB)
        mi = jnp.min(cand, axis=1, keepdims=True)
        pick = col == mi
        sel = jnp.logical_or(sel, pick)
        work = jnp.where(pick, neginf, work)

    s_full = jax.lax.dot_general(
        qt * _SCALE, kk, (((1,), (1,)), ((), ())),
        preferred_element_type=jnp.float32)
    selm = jnp.dot(sel.astype(jnp.float32), expand,
                   preferred_element_type=jnp.float32)
    p = _silu(s_full) * (selm * tri)
    o_slc = jnp.dot(p, vv, preferred_element_type=jnp.float32) * gs

    o_ref[0, 0] = o_cmp + o_slc


def _run(qh, kh, vh, gc, gs, tri, interpret=False):
    return pl.pallas_call(
        _fwd,
        grid=(_B, _H),
        in_specs=[
            pl.BlockSpec((1, 1, _L, _D), lambda b, h: (b, h, 0, 0)),
            pl.BlockSpec((1, 1, _L, _D), lambda b, h: (b, h, 0, 0)),
            pl.BlockSpec((1, 1, _L, _D), lambda b, h: (b, h, 0, 0)),
            pl.BlockSpec((1, 1, 1, _L, 1), lambda b, h: (b, h, 0, 0, 0)),
            pl.BlockSpec((1, 1, 1, _L, 1), lambda b, h: (b, h, 0, 0, 0)),
            pl.BlockSpec((1, 1, _L, _L), lambda b, h: (0, 0, 0, 0)),
        ],
        out_specs=pl.BlockSpec((1, 1, _L, _D), lambda b, h: (b, h, 0, 0)),
        out_shape=jax.ShapeDtypeStruct((_B, _H, _L, _D), jnp.float32),
        interpret=interpret,
    )(qh, kh, vh, gc, gs, tri)


def kernel(q, k, v, g_cmp, g_slc, x_offsets):
    del x_offsets
    qh = q.reshape(_B, _L, _H, _D).transpose(0, 2, 1, 3)
    kh = k.reshape(_B, _L, _H, _D).transpose(0, 2, 1, 3)
    vh = v.reshape(_B, _L, _H, _D).transpose(0, 2, 1, 3)
    gc = g_cmp.reshape(_B, _L, _H).transpose(0, 2, 1).reshape(_B, _H, 1, _L, 1)
    gs = g_slc.reshape(_B, _L, _H).transpose(0, 2, 1).reshape(_B, _H, 1, _L, 1)
    idx = jnp.arange(_L, dtype=jnp.int32)
    tri = (idx[None, :] <= idx[:, None]).astype(jnp.float32).reshape(1, 1, _L, _L)
    out = _run(qh, kh, vh, gc, gs, tri)
    return out.transpose(0, 2, 1, 3).reshape(_T, _H, _D)


# speedup vs baseline: 1.0585x; 1.0585x over previous
import jax
import jax.numpy as jnp
from jax.experimental import pallas as pl
from jax.experimental.pallas import tpu as pltpu

_B = 4
_L = 1024
_H = 4
_D = 32
_BS = 32
_S = 4
_T = _B * _L
_NB = _L // _BS
_SCALE = _D ** (-0.5)


def _silu(x):
    return x * jax.nn.sigmoid(x)


def _fwd(q_ref, k_ref, v_ref, gc_ref, gs_ref, o_ref):
    qt = q_ref[0, 0]
    kk = k_ref[0, 0]
    vv = v_ref[0, 0]
    gc = gc_ref[0, 0, 0]
    gs = gs_ref[0, 0, 0]

    blk_of_col = jax.lax.broadcasted_iota(jnp.int32, (_NB, _L), 1) // _BS
    blk_row = jax.lax.broadcasted_iota(jnp.int32, (_NB, _L), 0)
    expand = (blk_of_col == blk_row).astype(jnp.float32)

    k_cmp = kk.reshape(_NB, _BS, _D).sum(axis=1) * (1.0 / _BS)
    v_cmp = vv.reshape(_NB, _BS, _D).sum(axis=1) * (1.0 / _BS)

    s_cmp = jax.lax.dot_general(
        qt, k_cmp, (((1,), (1,)), ((), ())),
        preferred_element_type=jnp.float32) * _SCALE
    row = jax.lax.broadcasted_iota(jnp.int32, (_L, _NB), 0)
    col = jax.lax.broadcasted_iota(jnp.int32, (_L, _NB), 1)
    blk_causal = (row // _BS) >= col
    p_cmp = jnp.where(blk_causal, _silu(s_cmp), 0.0)
    o_cmp = jnp.dot(p_cmp, v_cmp, preferred_element_type=jnp.float32) * gc

    neginf = jnp.float32(-jnp.inf)
    work = jnp.where(blk_causal, s_cmp, neginf)
    sel = jnp.zeros((_L, _NB), dtype=jnp.bool_)
    for _ in range(_S):
        m = jnp.max(work, axis=1, keepdims=True)
        ismax = jnp.logical_and(work == m, jnp.logical_not(sel))
        cand = jnp.where(ismax, col, _NB)
        mi = jnp.min(cand, axis=1, keepdims=True)
        pick = col == mi
        sel = jnp.logical_or(sel, pick)
        work = jnp.where(pick, neginf, work)

    s_full = jax.lax.dot_general(
        qt * _SCALE, kk, (((1,), (1,)), ((), ())),
        preferred_element_type=jnp.float32)
    selm = jnp.dot(sel.astype(jnp.float32), expand,
                   preferred_element_type=jnp.float32)
    rowl = jax.lax.broadcasted_iota(jnp.int32, (_L, _L), 0)
    coll = jax.lax.broadcasted_iota(jnp.int32, (_L, _L), 1)
    tri = (coll <= rowl).astype(jnp.float32)
    p = _silu(s_full) * (selm * tri)
    o_slc = jnp.dot(p, vv, preferred_element_type=jnp.float32) * gs

    o_ref[0, 0] = o_cmp + o_slc


def _run(qh, kh, vh, gc, gs, interpret=False):
    return pl.pallas_call(
        _fwd,
        grid=(_B, _H),
        in_specs=[
            pl.BlockSpec((1, 1, _L, _D), lambda b, h: (b, h, 0, 0)),
            pl.BlockSpec((1, 1, _L, _D), lambda b, h: (b, h, 0, 0)),
            pl.BlockSpec((1, 1, _L, _D), lambda b, h: (b, h, 0, 0)),
            pl.BlockSpec((1, 1, 1, _L, 1), lambda b, h: (b, h, 0, 0, 0)),
            pl.BlockSpec((1, 1, 1, _L, 1), lambda b, h: (b, h, 0, 0, 0)),
        ],
        out_specs=pl.BlockSpec((1, 1, _L, _D), lambda b, h: (b, h, 0, 0)),
        out_shape=jax.ShapeDtypeStruct((_B, _H, _L, _D), jnp.float32),
        compiler_params=pltpu.CompilerParams(
            dimension_semantics=("parallel", "parallel")),
        interpret=interpret,
    )(qh, kh, vh, gc, gs)


def kernel(q, k, v, g_cmp, g_slc, x_offsets):
    del x_offsets
    qh = q.reshape(_B, _L, _H, _D).transpose(0, 2, 1, 3)
    kh = k.reshape(_B, _L, _H, _D).transpose(0, 2, 1, 3)
    vh = v.reshape(_B, _L, _H, _D).transpose(0, 2, 1, 3)
    gc = g_cmp.reshape(_B, _L, _H).transpose(0, 2, 1).reshape(_B, _H, 1, _L, 1)
    gs = g_slc.reshape(_B, _L, _H).transpose(0, 2, 1).reshape(_B, _H, 1, _L, 1)
    out = _run(qh, kh, vh, gc, gs)
    return out.transpose(0, 2, 1, 3).reshape(_T, _H, _D)


# speedup vs baseline: 1.2827x; 1.2118x over previous
import jax
import jax.numpy as jnp
from jax.experimental import pallas as pl
from jax.experimental.pallas import tpu as pltpu

_B = 4
_L = 1024
_H = 4
_D = 32
_BS = 32
_S = 4
_T = _B * _L
_NB = _L // _BS
_SCALE = _D ** (-0.5)


def _silu(x):
    return x * jax.nn.sigmoid(x)


def _fwd(q_ref, k_ref, v_ref, gc_ref, gs_ref, o_ref):
    qt = q_ref[0, 0]
    kk = k_ref[0, 0]
    vv = v_ref[0, 0]
    gc = gc_ref[0, 0, 0]
    gs = gs_ref[0, 0, 0]

    blk_of_col = jax.lax.broadcasted_iota(jnp.int32, (_NB, _L), 1) // _BS
    blk_row = jax.lax.broadcasted_iota(jnp.int32, (_NB, _L), 0)
    expand = (blk_of_col == blk_row).astype(jnp.float32)

    k_cmp = kk.reshape(_NB, _BS, _D).sum(axis=1) * (1.0 / _BS)
    v_cmp = vv.reshape(_NB, _BS, _D).sum(axis=1) * (1.0 / _BS)

    s_cmp = jax.lax.dot_general(
        k_cmp, qt, (((1,), (1,)), ((), ())),
        preferred_element_type=jnp.float32) * _SCALE
    tok_of_col = jax.lax.broadcasted_iota(jnp.int32, (_NB, _L), 1) // _BS
    blk_causal = tok_of_col >= blk_row
    p_cmp = jnp.where(blk_causal, _silu(s_cmp), 0.0)
    o_cmp = jax.lax.dot_general(
        p_cmp, v_cmp, (((0,), (0,)), ((), ())),
        preferred_element_type=jnp.float32) * gc

    neginf = jnp.float32(-jnp.inf)
    work = jnp.where(blk_causal, s_cmp, neginf)
    sel = jnp.zeros((_NB, _L), dtype=jnp.bool_)
    for _ in range(_S):
        m = jnp.max(work, axis=0, keepdims=True)
        ismax = jnp.logical_and(work == m, jnp.logical_not(sel))
        cand = jnp.where(ismax, blk_row, _NB)
        mi = jnp.min(cand, axis=0, keepdims=True)
        pick = blk_row == mi
        sel = jnp.logical_or(sel, pick)
        work = jnp.where(pick, neginf, work)

    s_full = jax.lax.dot_general(
        kk, qt * _SCALE, (((1,), (1,)), ((), ())),
        preferred_element_type=jnp.float32)
    selm = jax.lax.dot_general(
        expand, sel.astype(jnp.float32), (((0,), (0,)), ((), ())),
        preferred_element_type=jnp.float32)
    rowj = jax.lax.broadcasted_iota(jnp.int32, (_L, _L), 0)
    coll = jax.lax.broadcasted_iota(jnp.int32, (_L, _L), 1)
    keep = jnp.logical_and(selm > 0.5, rowj <= coll)
    p = jnp.where(keep, _silu(s_full), 0.0)
    o_slc = jax.lax.dot_general(
        p, vv, (((0,), (0,)), ((), ())),
        preferred_element_type=jnp.float32) * gs

    o_ref[0, 0] = o_cmp + o_slc


def _run(qh, kh, vh, gc, gs, interpret=False):
    return pl.pallas_call(
        _fwd,
        grid=(_B, _H),
        in_specs=[
            pl.BlockSpec((1, 1, _L, _D), lambda b, h: (b, h, 0, 0)),
            pl.BlockSpec((1, 1, _L, _D), lambda b, h: (b, h, 0, 0)),
            pl.BlockSpec((1, 1, _L, _D), lambda b, h: (b, h, 0, 0)),
            pl.BlockSpec((1, 1, 1, _L, 1), lambda b, h: (b, h, 0, 0, 0)),
            pl.BlockSpec((1, 1, 1, _L, 1), lambda b, h: (b, h, 0, 0, 0)),
        ],
        out_specs=pl.BlockSpec((1, 1, _L, _D), lambda b, h: (b, h, 0, 0)),
        out_shape=jax.ShapeDtypeStruct((_B, _H, _L, _D), jnp.float32),
        compiler_params=pltpu.CompilerParams(
            dimension_semantics=("parallel", "parallel")),
        interpret=interpret,
    )(qh, kh, vh, gc, gs)


def kernel(q, k, v, g_cmp, g_slc, x_offsets):
    del x_offsets
    qh = q.reshape(_B, _L, _H, _D).transpose(0, 2, 1, 3)
    kh = k.reshape(_B, _L, _H, _D).transpose(0, 2, 1, 3)
    vh = v.reshape(_B, _L, _H, _D).transpose(0, 2, 1, 3)
    gc = g_cmp.reshape(_B, _L, _H).transpose(0, 2, 1).reshape(_B, _H, 1, _L, 1)
    gs = g_slc.reshape(_B, _L, _H).transpose(0, 2, 1).reshape(_B, _H, 1, _L, 1)
    out = _run(qh, kh, vh, gc, gs)
    return out.transpose(0, 2, 1, 3).reshape(_T, _H, _D)


# speedup vs baseline: 1.3865x; 1.0810x over previous
import jax
import jax.numpy as jnp
from jax.experimental import pallas as pl
from jax.experimental.pallas import tpu as pltpu

_B = 4
_L = 1024
_H = 4
_D = 32
_BS = 32
_S = 4
_T = _B * _L
_NB = _L // _BS
_SCALE = _D ** (-0.5)


def _silu(x):
    return x * jax.nn.sigmoid(x)


def _fwd(q_ref, k_ref, v_ref, gc_ref, gs_ref, o_ref):
    qt = q_ref[0, 0]
    kk = k_ref[0, 0]
    vv = v_ref[0, 0]
    gc = gc_ref[0, 0, 0]
    gs = gs_ref[0, 0, 0]

    blk_row = jax.lax.broadcasted_iota(jnp.int32, (_NB, _L), 0)

    k_cmp = kk.reshape(_NB, _BS, _D).sum(axis=1) * (1.0 / _BS)
    v_cmp = vv.reshape(_NB, _BS, _D).sum(axis=1) * (1.0 / _BS)

    s_cmp = jax.lax.dot_general(
        k_cmp, qt, (((1,), (1,)), ((), ())),
        preferred_element_type=jnp.float32) * _SCALE
    tok_of_col = jax.lax.broadcasted_iota(jnp.int32, (_NB, _L), 1) // _BS
    blk_causal = tok_of_col >= blk_row
    p_cmp = jnp.where(blk_causal, _silu(s_cmp), 0.0)
    o_cmp = jax.lax.dot_general(
        p_cmp, v_cmp, (((0,), (0,)), ((), ())),
        preferred_element_type=jnp.float32) * gc

    neginf = jnp.float32(-jnp.inf)
    work = jnp.where(blk_causal, s_cmp, neginf)
    sel = jnp.zeros((_NB, _L), dtype=jnp.bool_)
    for _ in range(_S):
        m = jnp.max(work, axis=0, keepdims=True)
        ismax = jnp.logical_and(work == m, jnp.logical_not(sel))
        cand = jnp.where(ismax, blk_row, _NB)
        mi = jnp.min(cand, axis=0, keepdims=True)
        pick = blk_row == mi
        sel = jnp.logical_or(sel, pick)
        work = jnp.where(pick, neginf, work)

    s_full = jax.lax.dot_general(
        kk, qt * _SCALE, (((1,), (1,)), ((), ())),
        preferred_element_type=jnp.float32)
    sel_rep = jnp.repeat(sel, _BS, axis=0)
    rowj = jax.lax.broadcasted_iota(jnp.int32, (_L, _L), 0)
    coll = jax.lax.broadcasted_iota(jnp.int32, (_L, _L), 1)
    keep = jnp.logical_and(sel_rep, rowj <= coll)
    p = jnp.where(keep, _silu(s_full), 0.0)
    o_slc = jax.lax.dot_general(
        p, vv, (((0,), (0,)), ((), ())),
        preferred_element_type=jnp.float32) * gs

    o_ref[0, 0] = o_cmp + o_slc


def _run(qh, kh, vh, gc, gs, interpret=False):
    return pl.pallas_call(
        _fwd,
        grid=(_B, _H),
        in_specs=[
            pl.BlockSpec((1, 1, _L, _D), lambda b, h: (b, h, 0, 0)),
            pl.BlockSpec((1, 1, _L, _D), lambda b, h: (b, h, 0, 0)),
            pl.BlockSpec((1, 1, _L, _D), lambda b, h: (b, h, 0, 0)),
            pl.BlockSpec((1, 1, 1, _L, 1), lambda b, h: (b, h, 0, 0, 0)),
            pl.BlockSpec((1, 1, 1, _L, 1), lambda b, h: (b, h, 0, 0, 0)),
        ],
        out_specs=pl.BlockSpec((1, 1, _L, _D), lambda b, h: (b, h, 0, 0)),
        out_shape=jax.ShapeDtypeStruct((_B, _H, _L, _D), jnp.float32),
        compiler_params=pltpu.CompilerParams(
            dimension_semantics=("parallel", "parallel")),
        interpret=interpret,
    )(qh, kh, vh, gc, gs)


def kernel(q, k, v, g_cmp, g_slc, x_offsets):
    del x_offsets
    qh = q.reshape(_B, _L, _H, _D).transpose(0, 2, 1, 3)
    kh = k.reshape(_B, _L, _H, _D).transpose(0, 2, 1, 3)
    vh = v.reshape(_B, _L, _H, _D).transpose(0, 2, 1, 3)
    gc = g_cmp.reshape(_B, _L, _H).transpose(0, 2, 1).reshape(_B, _H, 1, _L, 1)
    gs = g_slc.reshape(_B, _L, _H).transpose(0, 2, 1).reshape(_B, _H, 1, _L, 1)
    out = _run(qh, kh, vh, gc, gs)
    return out.transpose(0, 2, 1, 3).reshape(_T, _H, _D)
